# Initial kernel scaffold; baseline (speedup 1.0000x reference)
import functools

import jax
import jax.numpy as jnp
from jax import lax
from jax.experimental import pallas as pl
from jax.experimental.pallas import tpu as pltpu
from jax.experimental.pallas import tpu_sc as plsc

_NC = 2
_NS = 16
_L = 16
_NW = _NC * _NS

_CH = 112


def _stage1_body(S, S_PAD, nx, ny, nz, tv_ref, p_ref, idx_ref, w_ref, nc_ref):
    tv = tv_ref[...]
    p = p_ref[...]
    t0 = tv[:, :-1]
    t1 = tv[:, 1:]
    fin = jnp.isfinite(t0) & jnp.isfinite(t1)
    t0s = jnp.where(fin, t0, 0.0)
    t1s = jnp.where(fin, t1, 0.0)
    valid = fin & (t1 > t0)
    tmid = 0.5 * (t0s + t1s)
    seg = (t1s - t0s) * p[:, 6:7]
    vx = p[:, 0:1] + tmid * p[:, 3:4]
    vy = p[:, 1:2] + tmid * p[:, 4:5]
    vz = p[:, 2:3] + tmid * p[:, 5:6]
    ix = jnp.floor(vx).astype(jnp.int32)
    iy = jnp.floor(vy).astype(jnp.int32)
    iz = jnp.floor(vz).astype(jnp.int32)
    inb = ((ix >= 0) & (ix < nx) & (iy >= 0) & (iy < ny)
           & (iz >= 0) & (iz < nz))
    ixc = jnp.clip(ix, 0, nx - 1)
    iyc = jnp.clip(iy, 0, ny - 1)
    izc = jnp.clip(iz, 0, nz - 1)
    lin = (ixc * ny + iyc) * nz + izc
    w = jnp.where(valid & inb, seg, 0.0)
    rb = tv.shape[0]
    pad = S_PAD - S
    idx_ref[...] = jnp.concatenate(
        [lin, jnp.zeros((rb, pad), jnp.int32)], axis=1)
    w_ref[...] = jnp.concatenate(
        [w, jnp.zeros((rb, pad), jnp.float32)], axis=1)
    nseg = jnp.sum(fin.astype(jnp.int32), axis=1, keepdims=True)
    nc_ref[...] = (nseg + (_CH - 1)) // _CH


def _sc_body(R, S_PAD, rpw, nch,
             idx_hbm, w_hbm, vol_hbm, out_hbm,
             idx_v, w_v, g_v, out_v, sem):
    wid = lax.axis_index("s") * _NC + lax.axis_index("c")
    base = wid * rpw
    zeros = jnp.zeros((_L,), jnp.float32)
    for j in range(nch):
        for k in range(_CH // _L):
            g_v[j, pl.ds(k * _L, _L)] = zeros

    def ray_body(r, carry):
        row = base + r
        pltpu.sync_copy(idx_hbm.at[row], idx_v)
        pltpu.sync_copy(w_hbm.at[row], w_v)
        for j in range(nch):
            pltpu.async_copy(vol_hbm.at[idx_v.at[j]], g_v.at[j], sem)
        for j in range(nch):
            pltpu.make_async_copy(vol_hbm.at[idx_v.at[j]], g_v.at[j], sem).wait()
        acc = jnp.zeros((_L,), jnp.float32)
        for j in range(nch):
            for k in range(_CH // _L):
                acc = acc + (g_v[j, pl.ds(k * _L, _L)]
                             * w_v[pl.ds(j * _CH + k * _L, _L)])
        out_v[r] = jnp.sum(acc)
        return carry

    lax.fori_loop(0, rpw, ray_body, 0)
    pltpu.sync_copy(out_v, out_hbm.at[pl.ds(base, rpw)])


def kernel(volume, tvals, M, b, src, dst):
    nx, ny, nz = volume.shape
    R, Sp1 = tvals.shape
    S = Sp1 - 1
    nch = -(-S // _CH)
    S_PAD = nch * _CH
    rpw = R // _NW

    d = dst - src
    a = src @ M.T + b
    c = d @ M.T
    ray_len = jnp.sqrt(jnp.sum(d * d, axis=1))
    p = jnp.concatenate(
        [a, c, ray_len[:, None], jnp.zeros((R, 1), jnp.float32)], axis=1)

    RB = 256
    grid = (R // RB,)
    idx, w, nc = pl.pallas_call(
        functools.partial(_stage1_body, S, S_PAD, nx, ny, nz),
        grid=grid,
        in_specs=[
            pl.BlockSpec((RB, Sp1), lambda i: (i, 0)),
            pl.BlockSpec((RB, 8), lambda i: (i, 0)),
        ],
        out_specs=[
            pl.BlockSpec((RB, S_PAD), lambda i: (i, 0)),
            pl.BlockSpec((RB, S_PAD), lambda i: (i, 0)),
            pl.BlockSpec((RB, 1), lambda i: (i, 0)),
        ],
        out_shape=[
            jax.ShapeDtypeStruct((R, S_PAD), jnp.int32),
            jax.ShapeDtypeStruct((R, S_PAD), jnp.float32),
            jax.ShapeDtypeStruct((R, 1), jnp.int32),
        ],
    )(tvals, p)
    del nc

    idx3 = idx.reshape(R, nch, _CH)
    vol_flat = volume.reshape(-1)

    mesh = plsc.VectorSubcoreMesh(core_axis_name="c", subcore_axis_name="s")
    sino = pl.kernel(
        functools.partial(_sc_body, R, S_PAD, rpw, nch),
        out_type=jax.ShapeDtypeStruct((R,), jnp.float32),
        mesh=mesh,
        scratch_types=[
            pltpu.VMEM((nch, _CH), jnp.int32),
            pltpu.VMEM((S_PAD,), jnp.float32),
            pltpu.VMEM((nch, _CH), jnp.float32),
            pltpu.VMEM((rpw,), jnp.float32),
            pltpu.SemaphoreType.DMA,
        ],
    )(idx3, w, vol_flat)
    return sino


# speedup vs baseline: 8.7409x; 8.7409x over previous
import functools

import jax
import jax.numpy as jnp
from jax import lax
from jax.experimental import pallas as pl
from jax.experimental.pallas import tpu as pltpu
from jax.experimental.pallas import tpu_sc as plsc

_NC = 2
_NS = 16
_L = 16
_NW = _NC * _NS

_CH = 112


def _stage1_body(S, S_PAD, nx, ny, nz, tv_ref, p_ref, mb_ref, idx_ref, w_ref,
                 nc_ref):
    tv = tv_ref[...]
    p = p_ref[...]
    t0 = tv[:, :-1]
    t1 = tv[:, 1:]
    fin = jnp.isfinite(t0) & jnp.isfinite(t1)
    t0s = jnp.where(fin, t0, 0.0)
    t1s = jnp.where(fin, t1, 0.0)
    valid = fin & (t1 > t0)
    tmid = 0.5 * (t0s + t1s)
    seg = (t1s - t0s) * p[:, 6:7]
    def q(x):
        return x.astype(jnp.bfloat16).astype(jnp.float32)
    px = q(p[:, 0:1] + tmid * p[:, 3:4])
    py = q(p[:, 1:2] + tmid * p[:, 4:5])
    pz = q(p[:, 2:3] + tmid * p[:, 5:6])
    vx = px * mb_ref[0] + py * mb_ref[1] + pz * mb_ref[2] + mb_ref[9]
    vy = px * mb_ref[3] + py * mb_ref[4] + pz * mb_ref[5] + mb_ref[10]
    vz = px * mb_ref[6] + py * mb_ref[7] + pz * mb_ref[8] + mb_ref[11]
    ix = jnp.floor(vx).astype(jnp.int32)
    iy = jnp.floor(vy).astype(jnp.int32)
    iz = jnp.floor(vz).astype(jnp.int32)
    inb = ((ix >= 0) & (ix < nx) & (iy >= 0) & (iy < ny)
           & (iz >= 0) & (iz < nz))
    ixc = jnp.clip(ix, 0, nx - 1)
    iyc = jnp.clip(iy, 0, ny - 1)
    izc = jnp.clip(iz, 0, nz - 1)
    lin = (ixc * ny + iyc) * nz + izc
    w = jnp.where(valid & inb, seg, 0.0)
    rb = tv.shape[0]
    pad = S_PAD - S
    idx_ref[...] = jnp.concatenate(
        [lin, jnp.zeros((rb, pad), jnp.int32)], axis=1)
    w_ref[...] = jnp.concatenate(
        [w, jnp.zeros((rb, pad), jnp.float32)], axis=1)
    nseg = jnp.sum(fin.astype(jnp.int32), axis=1, keepdims=True)
    nc_ref[...] = (nseg + (_CH - 1)) // _CH


def _sc_body(R, S_PAD, rpw, nch,
             idx_hbm, w_hbm, nc_hbm, vol_hbm, out_hbm,
             idx_v, w_v, g_v, nc_v, sums_v, acc_v, sem):
    wid = lax.axis_index("s") * _NC + lax.axis_index("c")
    base = wid * rpw
    pltpu.sync_copy(nc_hbm.at[pl.ds(base, rpw)], nc_v)
    zeros = jnp.zeros((_L,), jnp.float32)
    for j in range(nch):
        for k in range(_CH // _L):
            g_v[j, pl.ds(k * _L, _L)] = zeros
    lane = lax.iota(jnp.int32, _L)
    ngroups = rpw // _L

    def group_body(g, carry):
        ncg = nc_v[pl.ds(g * _L, _L)]
        for k in range(_L):
            r = g * _L + k
            row = base + r
            pltpu.sync_copy(idx_hbm.at[row], idx_v)
            pltpu.sync_copy(w_hbm.at[row], w_v)
            nc = ncg[k]
            for j in range(nch):
                @pl.when(j < nc)
                def _():
                    pltpu.async_copy(vol_hbm.at[idx_v.at[j]], g_v.at[j], sem)
            for j in range(nch):
                @pl.when(j < nc)
                def _():
                    pltpu.make_async_copy(
                        vol_hbm.at[idx_v.at[j]], g_v.at[j], sem).wait()
            acc = jnp.zeros((_L,), jnp.float32)
            for j in range(nch):
                for k2 in range(_CH // _L):
                    acc = acc + (g_v[j, pl.ds(k2 * _L, _L)]
                                 * w_v[pl.ds(j * _CH + k2 * _L, _L)])
            acc_v[pl.ds(k * _L, _L)] = acc
        tot = jnp.zeros((_L,), jnp.float32)
        for j in range(_L):
            tot = tot + plsc.load_gather(acc_v, [lane * _L + j])
        sums_v[...] = tot
        pltpu.sync_copy(sums_v, out_hbm.at[pl.ds(base + g * _L, _L)])
        return carry

    lax.fori_loop(0, ngroups, group_body, 0)


def kernel(volume, tvals, M, b, src, dst):
    nx, ny, nz = volume.shape
    R, Sp1 = tvals.shape
    S = Sp1 - 1
    nch = -(-S // _CH)
    S_PAD = nch * _CH
    rpw = R // _NW

    d = dst - src
    ray_len = jnp.sqrt(jnp.sum(d * d, axis=1))
    p = jnp.concatenate(
        [src, d, ray_len[:, None], jnp.zeros((R, 1), jnp.float32)], axis=1)
    mq = M.astype(jnp.bfloat16).astype(jnp.float32)
    mb = jnp.concatenate([mq.reshape(9), b, jnp.zeros((4,), jnp.float32)])

    RB = 256
    grid = (R // RB,)
    idx, w, nc = pl.pallas_call(
        functools.partial(_stage1_body, S, S_PAD, nx, ny, nz),
        grid=grid,
        in_specs=[
            pl.BlockSpec((RB, Sp1), lambda i: (i, 0)),
            pl.BlockSpec((RB, 8), lambda i: (i, 0)),
            pl.BlockSpec(memory_space=pltpu.SMEM),
        ],
        out_specs=[
            pl.BlockSpec((RB, S_PAD), lambda i: (i, 0)),
            pl.BlockSpec((RB, S_PAD), lambda i: (i, 0)),
            pl.BlockSpec((RB, 1), lambda i: (i, 0)),
        ],
        out_shape=[
            jax.ShapeDtypeStruct((R, S_PAD), jnp.int32),
            jax.ShapeDtypeStruct((R, S_PAD), jnp.float32),
            jax.ShapeDtypeStruct((R, 1), jnp.int32),
        ],
    )(tvals, p, mb)

    idx3 = idx.reshape(R, nch, _CH)
    nc1 = nc.reshape(R)
    vol_flat = volume.reshape(-1)

    mesh = plsc.VectorSubcoreMesh(core_axis_name="c", subcore_axis_name="s")
    sino = pl.kernel(
        functools.partial(_sc_body, R, S_PAD, rpw, nch),
        out_type=jax.ShapeDtypeStruct((R,), jnp.float32),
        mesh=mesh,
        compiler_params=pltpu.CompilerParams(needs_layout_passes=False),
        scratch_types=[
            pltpu.VMEM((nch, _CH), jnp.int32),
            pltpu.VMEM((S_PAD,), jnp.float32),
            pltpu.VMEM((nch, _CH), jnp.float32),
            pltpu.VMEM((rpw,), jnp.int32),
            pltpu.VMEM((_L,), jnp.float32),
            pltpu.VMEM((_L * _L,), jnp.float32),
            pltpu.SemaphoreType.DMA,
        ],
    )(idx3, w, nc1, vol_flat)
    return sino
